# baseline (speedup 1.0000x reference)
import functools

import jax
import jax.numpy as jnp
from jax import lax
from jax.experimental import pallas as pl
from jax.experimental.pallas import tpu as pltpu
from jax.experimental.pallas import tpu_sc as plsc

VOCAB = 100000
B = 4096
S = 200
D = 64
NC, NS = 2, 16
NW = NC * NS
BB = 128
NB = B // BB
LANES = 16
DT = D // 8
OPAD = BB + 1


def kernel(x, token_table, pos_table):
    xt3 = x.astype(jnp.int32).T.reshape(S, NB, BB)
    mesh = plsc.VectorSubcoreMesh(core_axis_name="c", subcore_axis_name="s")

    @functools.partial(
        pl.kernel,
        out_type=jax.ShapeDtypeStruct((S, DT, NB, 8, BB), jnp.float32),
        mesh=mesh,
        compiler_params=pltpu.CompilerParams(use_tc_tiling_on_sc=False,
                                             needs_layout_passes=False),
        scratch_types=[
            pltpu.VMEM((S, BB), jnp.int32),
            pltpu.VMEM((S, D), jnp.float32),
            pltpu.VMEM((BB, D), jnp.float32),
            pltpu.VMEM((BB, D), jnp.float32),
            pltpu.VMEM((BB, D), jnp.float32),
            pltpu.VMEM((BB, D), jnp.float32),
            pltpu.VMEM((BB, D), jnp.float32),
            pltpu.VMEM((DT, 8, OPAD), jnp.float32),
            pltpu.VMEM((DT, 8, OPAD), jnp.float32),
            pltpu.VMEM((DT, 8, OPAD), jnp.float32),
            pltpu.VMEM((DT, 8, OPAD), jnp.float32),
            pltpu.VMEM((DT, 8, OPAD), jnp.float32),
            pltpu.SemaphoreType.DMA,
            pltpu.SemaphoreType.DMA,
            pltpu.SemaphoreType.DMA,
            pltpu.SemaphoreType.DMA,
            pltpu.SemaphoreType.DMA,
            pltpu.SemaphoreType.DMA,
            pltpu.SemaphoreType.DMA,
            pltpu.SemaphoreType.DMA,
            pltpu.SemaphoreType.DMA,
            pltpu.SemaphoreType.DMA,
        ],
    )
    def run(x_ref, tok_ref, pos_ref, out_ref,
            idx_v, pos_v, tok_v0, tok_v1, tok_v2, tok_v3, tok_v4,
            out_v0, out_v1, out_v2, out_v3, out_v4,
            gsem0, gsem1, gsem2, gsem3, gsem4,
            osem0, osem1, osem2, osem3, osem4):
        tok_v = (tok_v0, tok_v1, tok_v2, tok_v3, tok_v4)
        out_v = (out_v0, out_v1, out_v2, out_v3, out_v4)
        gsem = (gsem0, gsem1, gsem2, gsem3, gsem4)
        osem = (osem0, osem1, osem2, osem3, osem4)
        NBUF = 5

        wid = lax.axis_index("s") * NC + lax.axis_index("c")
        pltpu.sync_copy(pos_ref, pos_v)
        pltpu.sync_copy(x_ref.at[:, wid], idx_v)
        iota = lax.iota(jnp.int32, LANES)
        dt_c = [(iota + dg * LANES) // 8 for dg in range(D // LANES)]
        dr_c = [(iota + dg * LANES) % 8 for dg in range(D // LANES)]

        def wb_start(s, b):
            pltpu.async_copy(out_v[b].at[:, :, pl.ds(0, BB)],
                             out_ref.at[s, :, wid], osem[b])

        def wb_wait(s, b):
            pltpu.make_async_copy(out_v[b].at[:, :, pl.ds(0, BB)],
                                  out_ref.at[s, :, wid], osem[b]).wait()

        for b in range(NBUF):
            pltpu.async_copy(tok_ref.at[idx_v.at[b]], tok_v[b], gsem[b])

        @pl.loop(0, S, step=NBUF)
        def _pair(sp):
            for b in range(NBUF):
                s = sp + b
                pltpu.make_async_copy(tok_ref.at[idx_v.at[s]], tok_v[b],
                                      gsem[b]).wait()

                @pl.when(s >= NBUF)
                def _():
                    wb_wait(s - NBUF, b)

                pos_c = [pos_v[s, pl.ds(dg * LANES, LANES)]
                         for dg in range(D // LANES)]

                @plsc.parallel_loop(0, BB, unroll=4)
                def _row(bb):
                    cols = jnp.full((LANES,), bb, jnp.int32)
                    for dg in range(D // LANES):
                        vals = tok_v[b][bb, pl.ds(dg * LANES, LANES)]
                        plsc.store_scatter(out_v[b], [dt_c[dg], dr_c[dg], cols],
                                           vals + pos_c[dg])

                wb_start(s, b)

                @pl.when(s + NBUF < S)
                def _():
                    pltpu.async_copy(tok_ref.at[idx_v.at[s + NBUF]], tok_v[b],
                                     gsem[b])

        for b in range(NBUF):
            wb_wait(S - NBUF + b, b)

    out5 = run(xt3, token_table, pos_table)
    return out5.transpose(2, 4, 0, 1, 3).reshape(B, S, D)


# speedup vs baseline: 1.0011x; 1.0011x over previous
import functools

import jax
import jax.numpy as jnp
from jax import lax
from jax.experimental import pallas as pl
from jax.experimental.pallas import tpu as pltpu
from jax.experimental.pallas import tpu_sc as plsc

VOCAB = 100000
B = 4096
S = 200
D = 64
NC, NS = 2, 16
NW = NC * NS
BB = 128
NB = B // BB
LANES = 16
DT = D // 8
OPAD = BB + 1


def kernel(x, token_table, pos_table):
    xt3 = x.astype(jnp.int32).T.reshape(S, NB, BB)
    mesh = plsc.VectorSubcoreMesh(core_axis_name="c", subcore_axis_name="s")

    @functools.partial(
        pl.kernel,
        out_type=jax.ShapeDtypeStruct((S, DT, NB, 8, BB), jnp.float32),
        mesh=mesh,
        compiler_params=pltpu.CompilerParams(use_tc_tiling_on_sc=False,
                                             needs_layout_passes=False),
        scratch_types=[
            pltpu.VMEM((S, BB), jnp.int32),
            pltpu.VMEM((S, D), jnp.float32),
            pltpu.VMEM((BB, D), jnp.float32),
            pltpu.VMEM((BB, D), jnp.float32),
            pltpu.VMEM((BB, D), jnp.float32),
            pltpu.VMEM((BB, D), jnp.float32),
            pltpu.VMEM((DT, 8, OPAD), jnp.float32),
            pltpu.VMEM((DT, 8, OPAD), jnp.float32),
            pltpu.VMEM((DT, 8, OPAD), jnp.float32),
            pltpu.VMEM((DT, 8, OPAD), jnp.float32),
            pltpu.SemaphoreType.DMA,
            pltpu.SemaphoreType.DMA,
            pltpu.SemaphoreType.DMA,
            pltpu.SemaphoreType.DMA,
            pltpu.SemaphoreType.DMA,
            pltpu.SemaphoreType.DMA,
            pltpu.SemaphoreType.DMA,
            pltpu.SemaphoreType.DMA,
        ],
    )
    def run(x_ref, tok_ref, pos_ref, out_ref,
            idx_v, pos_v, tok_v0, tok_v1, tok_v2, tok_v3,
            out_v0, out_v1, out_v2, out_v3,
            gsem0, gsem1, gsem2, gsem3, osem0, osem1, osem2, osem3):
        tok_v = (tok_v0, tok_v1, tok_v2, tok_v3)
        out_v = (out_v0, out_v1, out_v2, out_v3)
        gsem = (gsem0, gsem1, gsem2, gsem3)
        osem = (osem0, osem1, osem2, osem3)
        NBUF = 4

        wid = lax.axis_index("s") * NC + lax.axis_index("c")
        pltpu.sync_copy(pos_ref, pos_v)
        pltpu.sync_copy(x_ref.at[:, wid], idx_v)
        iota = lax.iota(jnp.int32, LANES)
        dt_c = [(iota + dg * LANES) // 8 for dg in range(D // LANES)]
        dr_c = [(iota + dg * LANES) % 8 for dg in range(D // LANES)]

        def wb_start(s, b):
            pltpu.async_copy(out_v[b].at[:, :, pl.ds(0, BB)],
                             out_ref.at[s, :, wid], osem[b])

        def wb_wait(s, b):
            pltpu.make_async_copy(out_v[b].at[:, :, pl.ds(0, BB)],
                                  out_ref.at[s, :, wid], osem[b]).wait()

        for b in range(NBUF):
            pltpu.async_copy(tok_ref.at[idx_v.at[b]], tok_v[b], gsem[b])

        @pl.loop(0, S, step=NBUF)
        def _pair(sp):
            for b in range(NBUF):
                s = sp + b
                pltpu.make_async_copy(tok_ref.at[idx_v.at[s]], tok_v[b],
                                      gsem[b]).wait()

                @pl.when(s >= NBUF)
                def _():
                    wb_wait(s - NBUF, b)

                pos_c = [pos_v[s, pl.ds(dg * LANES, LANES)]
                         for dg in range(D // LANES)]

                @plsc.parallel_loop(0, BB, unroll=4)
                def _row(bb):
                    cols = jnp.full((LANES,), bb, jnp.int32)
                    for dg in range(D // LANES):
                        vals = tok_v[b][bb, pl.ds(dg * LANES, LANES)]
                        plsc.store_scatter(out_v[b], [dt_c[dg], dr_c[dg], cols],
                                           vals + pos_c[dg])

                wb_start(s, b)

                @pl.when(s + NBUF < S)
                def _():
                    pltpu.async_copy(tok_ref.at[idx_v.at[s + NBUF]], tok_v[b],
                                     gsem[b])

        for b in range(NBUF):
            wb_wait(S - NBUF + b, b)

    out5 = run(xt3, token_table, pos_table)
    return out5.transpose(2, 4, 0, 1, 3).reshape(B, S, D)
